# baseline (speedup 1.0000x reference)
import functools

import jax
import jax.numpy as jnp
from jax import lax
from jax.experimental import pallas as pl
from jax.experimental.pallas import tpu as pltpu
from jax.experimental.pallas import tpu_sc as plsc

D = 256
CHUNK = 128
LANES = 16
MAXWIN = 26
NB = 26
NBPAD = 32


@functools.partial(jax.jit, static_argnums=(0, 1, 2, 3))
def _build(rows_out, rows_in, nw, nchunk, h, idx32, cf):
    per = (-(-rows_out // nw) + 7) // 8 * 8

    mesh = plsc.VectorSubcoreMesh(core_axis_name="c", subcore_axis_name="s")
    nc = mesh.num_cores

    @functools.partial(
        pl.kernel,
        out_type=jax.ShapeDtypeStruct((rows_out, D), jnp.float32),
        mesh=mesh,
        scratch_types=[
            pltpu.VMEM((CHUNK, D), jnp.float32),
            pltpu.VMEM((2, CHUNK, D), jnp.float32),
            pltpu.VMEM((MAXWIN, CHUNK), jnp.int32),
            pltpu.VMEM((NBPAD,), jnp.int32),
            pltpu.SemaphoreType.DMA,
            pltpu.SemaphoreType.DMA,
            pltpu.SemaphoreType.DMA,
            pltpu.SemaphoreType.DMA,
        ],
    )
    def unpool(h_hbm, idx_hbm, cf_hbm, out_hbm,
               zeros_v, rows2_v, idx2_v, cf_v, semz, semi, semh, sems):
        w = lax.axis_index("s") * nc + lax.axis_index("c")

        def zbody(i, carry):
            r = i // (D // LANES)
            c = (i % (D // LANES)) * LANES
            zeros_v[r, pl.ds(c, LANES)] = jnp.zeros((LANES,), jnp.float32)
            return carry

        lax.fori_loop(0, CHUNK * (D // LANES), zbody, 0)

        pltpu.sync_copy(cf_hbm.at[pl.ds(w * NBPAD, NBPAD)], cf_v)
        v1 = cf_v[pl.ds(0, LANES)]
        v2 = cf_v[pl.ds(LANES, LANES)]
        cfv = [v1[j] if j < LANES else v2[j - LANES] for j in range(NB + 1)]

        lo = w * per
        hi = jnp.minimum(lo + per, rows_out)
        nfull = (hi - lo) // CHUNK
        s = cfv[0]
        e = cfv[NB - 1]

        nz = jnp.int32(0)
        for j in range(NB - 1):
            cond = jnp.logical_and(j < nfull, cfv[j + 1] - cfv[j] < CHUNK)

            @pl.when(cond)
            def _(j=j):
                pltpu.make_async_copy(
                    zeros_v, out_hbm.at[pl.ds(lo + j * CHUNK, CHUNK)], semz
                ).start()

            nz = nz + cond.astype(jnp.int32)
        cond_t = cfv[NB - 1] - cfv[NB] < CHUNK

        @pl.when(cond_t)
        def _():
            pltpu.make_async_copy(
                zeros_v, out_hbm.at[pl.ds(hi - CHUNK, CHUNK)], semz
            ).start()

        nz = nz + cond_t.astype(jnp.int32)

        a0 = (s // 8) * 8
        nwin = (e - a0 + CHUNK - 1) // CHUNK

        def astart(j):
            return jnp.minimum(a0 + j * CHUNK, rows_in - CHUNK)

        def iissue(j, carry):
            pltpu.make_async_copy(
                idx_hbm.at[pl.ds(astart(j), CHUNK)], idx2_v.at[j], semi
            ).start()
            return carry

        lax.fori_loop(0, nwin, iissue, 0)

        @pl.when(nwin >= 1)
        def _():
            pltpu.make_async_copy(
                h_hbm.at[pl.ds(astart(0), CHUNK)], rows2_v.at[0], semh
            ).start()

        def zdrain(j, carry):
            pltpu.make_async_copy(
                zeros_v, out_hbm.at[pl.ds(lo, CHUNK)], semz
            ).wait()
            return carry

        lax.fori_loop(0, nz, zdrain, 0)

        def idrain(j, carry):
            pltpu.make_async_copy(
                idx_hbm.at[pl.ds(0, CHUNK)], idx2_v.at[0], semi
            ).wait()
            return carry

        lax.fori_loop(0, nwin, idrain, 0)

        def scat(j, carry):
            b = j % 2
            pltpu.make_async_copy(
                h_hbm.at[pl.ds(0, CHUNK)], rows2_v.at[0], semh
            ).wait()

            @pl.when(j >= 1)
            def _():
                pltpu.make_async_copy(
                    rows2_v.at[0], out_hbm.at[idx2_v.at[0]], sems
                ).wait()

            @pl.when(j + 1 < nwin)
            def _():
                pltpu.make_async_copy(
                    h_hbm.at[pl.ds(astart(j + 1), CHUNK)], rows2_v.at[1 - b], semh
                ).start()

            pltpu.make_async_copy(
                rows2_v.at[b], out_hbm.at[idx2_v.at[j]], sems
            ).start()
            return carry

        lax.fori_loop(0, nwin, scat, 0)

        @pl.when(nwin >= 1)
        def _():
            pltpu.make_async_copy(
                rows2_v.at[0], out_hbm.at[idx2_v.at[0]], sems
            ).wait()

    return unpool(h, idx32, cf)


def kernel(g, h, idx):
    rows_out = g.shape[0]
    rows_in = h.shape[0]
    info = plsc.get_sparse_core_info()
    nw = info.num_cores * info.num_subcores

    idx32 = idx.astype(jnp.int32)
    per = (-(-rows_out // nw) + 7) // 8 * 8
    nchunk = per // CHUNK + 1

    wids = jnp.arange(nw)[:, None]
    lo_w = wids * per
    hi_w = jnp.minimum(lo_w + per, rows_out)
    bounds = jnp.minimum(lo_w + jnp.arange(NB - 1)[None, :] * CHUNK, hi_w)
    bounds = jnp.concatenate(
        [bounds, hi_w, hi_w - CHUNK, jnp.zeros((nw, NBPAD - NB - 1), jnp.int32)],
        axis=1,
    )
    cf = jnp.searchsorted(idx32, bounds.reshape(-1)).astype(jnp.int32)

    return _build(rows_out, rows_in, nw, nchunk, h, idx32, cf)
